# baseline (speedup 1.0000x reference)
import functools

import jax
import jax.numpy as jnp
from jax import lax
from jax.experimental import pallas as pl
from jax.experimental.pallas import tpu as pltpu
from jax.experimental.pallas import tpu_sc as plsc

PAD = 0
VOCAB = 100000
BATCH = 1024
NC, NS, L = 2, 16, 16
NW = NC * NS
ROWS_PER_W = BATCH // NW
FLAT = BATCH * VOCAB
WORK_PER_W = FLAT // NW
ZWORDS = 50000
ZS = ZWORDS * NS
NFILL = WORK_PER_W // ZS

_mesh = plsc.VectorSubcoreMesh(core_axis_name="c", subcore_axis_name="s")


@functools.partial(
    pl.kernel,
    out_type=jax.ShapeDtypeStruct((FLAT,), jnp.float32),
    mesh=_mesh,
    scratch_types=[
        pltpu.VMEM((ZWORDS,), jnp.float32),
        pltpu.MemorySpace.VMEM_SHARED((ZS,), jnp.float32),
        pltpu.VMEM((ROWS_PER_W,), jnp.int32),
        pltpu.VMEM((ROWS_PER_W,), jnp.int32),
        pltpu.VMEM((ROWS_PER_W,), jnp.float32),
        pltpu.SemaphoreType.DMA,
    ],
)
def _sc_onehot(ids_hbm, out_hbm, zbuf, zshared, ids_v, idx_v, val_v, sem):
    cid = lax.axis_index("c")
    sid = lax.axis_index("s")
    wid = cid * NS + sid
    base_row = wid * ROWS_PER_W
    flat_base = wid * WORK_PER_W

    zero16 = jnp.zeros((L,), jnp.float32)

    def zloop(i, carry):
        for u in range(5):
            zbuf[pl.ds((i * 5 + u) * L, L)] = zero16
        return carry

    lax.fori_loop(0, ZWORDS // (5 * L), zloop, 0)

    pltpu.sync_copy(zbuf, zshared.at[pl.ds(sid * ZWORDS, ZWORDS)])
    plsc.subcore_barrier()

    pltpu.sync_copy(ids_hbm.at[pl.ds(base_row, ROWS_PER_W)], ids_v)

    for k in range(NFILL):
        pltpu.async_copy(
            zshared, out_hbm.at[pl.ds(flat_base + k * ZS, ZS)], sem
        )

    iota = lax.iota(jnp.int32, L)
    for c in range(ROWS_PER_W // L):
        t = ids_v[pl.ds(c * L, L)]
        rows = base_row + c * L + iota
        idx_v[pl.ds(c * L, L)] = rows * VOCAB + t
        val_v[pl.ds(c * L, L)] = jnp.where(t != PAD, 1.0, 0.0).astype(
            jnp.float32
        )

    for k in range(NFILL):
        pltpu.make_async_copy(
            zshared, out_hbm.at[pl.ds(flat_base + k * ZS, ZS)], sem
        ).wait()

    pltpu.async_copy(val_v, out_hbm.at[idx_v], sem).wait()


@jax.jit
def kernel(trg_token_ids_batch):
    ids = trg_token_ids_batch.reshape(BATCH)
    out = _sc_onehot(ids)
    return out.reshape(BATCH, VOCAB)


# speedup vs baseline: 1.1932x; 1.1932x over previous
import functools

import jax
import jax.numpy as jnp
from jax import lax
from jax.experimental import pallas as pl
from jax.experimental.pallas import tpu as pltpu
from jax.experimental.pallas import tpu_sc as plsc

PAD = 0
VOCAB = 100000
BATCH = 1024
NC, NS, L = 2, 16, 16
NW = NC * NS
ROWS_PER_W = BATCH // NW
RG = 8
NG = ROWS_PER_W // RG
ALIGNED_COLS = 99968
ZW = 9088
NCH = ALIGNED_COLS // ZW
LAST_TILE = ALIGNED_COLS // 128
NGROUPS = BATCH // RG

_mesh = plsc.VectorSubcoreMesh(core_axis_name="c", subcore_axis_name="s")


@functools.partial(
    pl.kernel,
    out_type=jax.ShapeDtypeStruct((BATCH, VOCAB), jnp.float32),
    mesh=_mesh,
    scratch_types=[
        pltpu.VMEM((RG, ZW), jnp.float32),
        pltpu.SemaphoreType.DMA,
    ],
    compiler_params=pltpu.CompilerParams(use_tc_tiling_on_sc=True),
)
def _sc_fill(out_hbm, zbuf, sem):
    cid = lax.axis_index("c")
    sid = lax.axis_index("s")
    wid = cid * NS + sid
    base_row = pl.multiple_of(wid * ROWS_PER_W, RG)

    zero16 = jnp.zeros((L,), jnp.float32)

    def zloop(i, carry):
        for r in range(RG):
            zbuf[r, pl.ds(i * L, L)] = zero16
        return carry

    lax.fori_loop(0, ZW // L, zloop, 0)

    def _copies():
        for k in range(NG):
            rows = pl.ds(base_row + k * RG, RG)
            for c in range(NCH):
                yield zbuf, out_hbm.at[rows, pl.ds(c * ZW, ZW)]

    for src, dst in _copies():
        pltpu.async_copy(src, dst, sem)
    for src, dst in _copies():
        pltpu.make_async_copy(src, dst, sem).wait()


def _spike_body(ids_smem, ids_ref, _, out_ref):
    i = pl.program_id(0)
    tilecol = jnp.where(
        i < BATCH, ids_smem[jnp.minimum(i, BATCH - 1)] // 128, LAST_TILE
    )
    t = ids_ref[:]
    lanes = jax.lax.broadcasted_iota(jnp.int32, (RG, 128), 1)
    hit = (t // 128 == tilecol) & (lanes == t % 128) & (t != PAD)
    out_ref[:] = hit.astype(jnp.float32)


def _group_map(i, ids_ref):
    return jnp.where(i < BATCH, i // RG, i - BATCH)


@jax.jit
def kernel(trg_token_ids_batch):
    ids = trg_token_ids_batch.reshape(BATCH)
    filled = _sc_fill()
    grid_spec = pltpu.PrefetchScalarGridSpec(
        num_scalar_prefetch=1,
        grid=(BATCH + NGROUPS,),
        in_specs=[
            pl.BlockSpec((RG, 1), lambda i, ids_ref: (_group_map(i, ids_ref), 0)),
            pl.BlockSpec(memory_space=pl.MemorySpace.ANY),
        ],
        out_specs=pl.BlockSpec(
            (RG, 128),
            lambda i, ids_ref: (
                _group_map(i, ids_ref),
                jnp.where(
                    i < BATCH,
                    ids_ref[jnp.minimum(i, BATCH - 1)] // 128,
                    LAST_TILE,
                ),
            ),
        ),
    )
    out = pl.pallas_call(
        _spike_body,
        grid_spec=grid_spec,
        out_shape=jax.ShapeDtypeStruct((BATCH, VOCAB), jnp.float32),
        input_output_aliases={2: 0},
    )(ids, trg_token_ids_batch, filled)
    return out


# speedup vs baseline: 2.1824x; 1.8290x over previous
import functools

import jax
import jax.numpy as jnp
from jax import lax
from jax.experimental import pallas as pl
from jax.experimental.pallas import tpu as pltpu
from jax.experimental.pallas import tpu_sc as plsc

PAD = 0
VOCAB = 100000
BATCH = 1024
NC, NS, L = 2, 16, 16
NW = NC * NS
ROWS_PER_W = BATCH // NW
RG = 8
NG = ROWS_PER_W // RG
ALIGNED_COLS = 99968
ZW = 9088
NCH = ALIGNED_COLS // ZW
TAILC = ALIGNED_COLS
NFILL = NG * NCH

_mesh = plsc.VectorSubcoreMesh(core_axis_name="c", subcore_axis_name="s")


@functools.partial(
    pl.kernel,
    out_type=jax.ShapeDtypeStruct((BATCH, VOCAB), jnp.float32),
    mesh=_mesh,
    scratch_types=[
        pltpu.VMEM((RG, ZW), jnp.float32),
        pltpu.VMEM((ROWS_PER_W, RG, 128), jnp.float32),
        pltpu.VMEM((NG, RG, 32), jnp.float32),
        pltpu.VMEM((ROWS_PER_W,), jnp.int32),
        pltpu.SemaphoreType.DMA,
    ],
    compiler_params=pltpu.CompilerParams(
        use_tc_tiling_on_sc=True, needs_layout_passes=False
    ),
)
def _sc_onehot(ids_hbm, out_hbm, zbuf, stile, ttile, ids_v, sem):
    cid = lax.axis_index("c")
    sid = lax.axis_index("s")
    wid = cid * NS + sid
    base_row = pl.multiple_of(wid * ROWS_PER_W, RG)

    iota = lax.iota(jnp.int32, L)
    zero16 = jnp.zeros((L,), jnp.float32)

    def zloop(i, carry):
        for r in range(RG):
            zbuf[r, pl.ds(i * L, L)] = zero16
        return carry

    lax.fori_loop(0, ZW // L, zloop, 0)

    pltpu.sync_copy(ids_hbm.at[pl.ds(base_row, ROWS_PER_W)], ids_v)

    def fill_slices(k):
        g = k // NCH
        c = k % NCH
        rows = pl.ds(pl.multiple_of(base_row + g * RG, RG), RG)
        cols = pl.ds(pl.multiple_of(c * ZW, 128), ZW)
        return zbuf, out_hbm.at[rows, cols]

    def fire(k, carry):
        src, dst = fill_slices(k)
        pltpu.async_copy(src, dst, sem)
        return carry

    lax.fori_loop(0, NFILL, fire, 0)

    def _extract(tvec, j):
        return jnp.sum(jnp.where(iota == j, tvec, 0))

    def build(g, carry):
        tvec = ids_v[pl.ds(pl.multiple_of((g // 2) * L, L), L)]
        half = (g % 2) * RG
        ts = [_extract(tvec, half + q) for q in range(RG)]
        for r in range(RG):
            tilebase = (ts[r] // 128) * 128
            for q in range(RG):
                nz = ts[q] != PAD
                for c in range(128 // L):
                    hit = (ts[q] == tilebase + c * L + iota) & nz
                    stile[g * RG + r, q, pl.ds(c * L, L)] = jnp.where(
                        hit, 1.0, 0.0
                    ).astype(jnp.float32)
        for q in range(RG):
            nz = ts[q] != PAD
            for c in range(32 // L):
                hit = (ts[q] == TAILC + c * L + iota) & nz
                ttile[g, q, pl.ds(c * L, L)] = jnp.where(hit, 1.0, 0.0).astype(
                    jnp.float32
                )
        return carry

    lax.fori_loop(0, NG, build, 0)

    def drain(k, carry):
        src, dst = fill_slices(k)
        pltpu.make_async_copy(src, dst, sem).wait()
        return carry

    lax.fori_loop(0, NFILL, drain, 0)

    def spike_dma(g, carry):
        tvec = ids_v[pl.ds(pl.multiple_of((g // 2) * L, L), L)]
        half = (g % 2) * RG
        rows = pl.ds(pl.multiple_of(base_row + g * RG, RG), RG)
        for r in range(RG):
            t_r = _extract(tvec, half + r)
            colbase = pl.multiple_of((t_r // 128) * 128, 128)
            pltpu.async_copy(
                stile.at[g * RG + r],
                out_hbm.at[rows, pl.ds(colbase, 128)],
                sem,
            )
        pltpu.async_copy(
            ttile.at[g], out_hbm.at[rows, pl.ds(TAILC, 32)], sem
        )
        return carry

    lax.fori_loop(0, NG, spike_dma, 0)

    def spike_wait(g, carry):
        tvec = ids_v[pl.ds(pl.multiple_of((g // 2) * L, L), L)]
        half = (g % 2) * RG
        rows = pl.ds(pl.multiple_of(base_row + g * RG, RG), RG)
        for r in range(RG):
            t_r = _extract(tvec, half + r)
            colbase = pl.multiple_of((t_r // 128) * 128, 128)
            pltpu.make_async_copy(
                stile.at[g * RG + r],
                out_hbm.at[rows, pl.ds(colbase, 128)],
                sem,
            ).wait()
        pltpu.make_async_copy(
            ttile.at[g], out_hbm.at[rows, pl.ds(TAILC, 32)], sem
        ).wait()
        return carry

    lax.fori_loop(0, NG, spike_wait, 0)


@jax.jit
def kernel(trg_token_ids_batch):
    ids = trg_token_ids_batch.reshape(BATCH)
    return _sc_onehot(ids)


# speedup vs baseline: 2.1833x; 1.0004x over previous
import functools

import jax
import jax.numpy as jnp
from jax import lax
from jax.experimental import pallas as pl
from jax.experimental.pallas import tpu as pltpu
from jax.experimental.pallas import tpu_sc as plsc

PAD = 0
VOCAB = 100000
BATCH = 1024
NC, NS, L = 2, 16, 16
NW = NC * NS
ROWS_PER_W = BATCH // NW
RG = 8
NG = ROWS_PER_W // RG
ALIGNED_COLS = 99968
ZW = 9088
NCH = ALIGNED_COLS // ZW
NFILL = NG * NCH
MAXTILE = ALIGNED_COLS // 128 - 1

_mesh = plsc.VectorSubcoreMesh(core_axis_name="c", subcore_axis_name="s")


@functools.partial(
    pl.kernel,
    out_type=jax.ShapeDtypeStruct((BATCH, VOCAB), jnp.float32),
    mesh=_mesh,
    scratch_types=[
        pltpu.VMEM((RG, ZW), jnp.float32),
        pltpu.VMEM((ROWS_PER_W, RG, 128), jnp.float32),
        pltpu.VMEM((NG, RG, 32), jnp.float32),
        pltpu.VMEM((ROWS_PER_W,), jnp.int32),
        pltpu.SemaphoreType.DMA,
    ],
    compiler_params=pltpu.CompilerParams(
        use_tc_tiling_on_sc=True, needs_layout_passes=False
    ),
)
def _sc_onehot(ids_hbm, out_hbm, zbuf, stile, ttile, ids_v, sem):
    cid = lax.axis_index("c")
    sid = lax.axis_index("s")
    wid = cid * NS + sid
    base_row = pl.multiple_of(wid * ROWS_PER_W, RG)

    iota = lax.iota(jnp.int32, L)
    zero16 = jnp.zeros((L,), jnp.float32)
    one16 = jnp.ones((L,), jnp.float32)

    def zloop(i, carry):
        for r in range(RG):
            zbuf[r, pl.ds(i * L, L)] = zero16
        return carry

    lax.fori_loop(0, ZW // L, zloop, 0)

    pltpu.sync_copy(ids_hbm.at[pl.ds(base_row, ROWS_PER_W)], ids_v)

    def fill_slices(k):
        g = k // NCH
        c = k % NCH
        rows = pl.ds(pl.multiple_of(base_row + g * RG, RG), RG)
        cols = pl.ds(pl.multiple_of(c * ZW, 128), ZW)
        return zbuf, out_hbm.at[rows, cols]

    def fire(k, carry):
        src, dst = fill_slices(k)
        pltpu.async_copy(src, dst, sem)
        return carry

    lax.fori_loop(0, NFILL, fire, 0)

    def zs(i, carry):
        stile[i // 64, (i // 8) % RG, pl.ds((i % 8) * L, L)] = zero16
        return carry

    lax.fori_loop(0, ROWS_PER_W * RG * 128 // L, zs, 0)

    def zt(i, carry):
        ttile[i // 16, (i // 2) % RG, pl.ds((i % 2) * L, L)] = zero16
        return carry

    lax.fori_loop(0, NG * RG * 32 // L, zt, 0)

    def _extract(tvec, pos):
        return jnp.sum(jnp.where(iota == pos, tvec, 0))

    def build(r, carry):
        tvec = ids_v[pl.ds((r // L) * L, L)]
        pos = r % L
        t_r = _extract(tvec, pos)
        tilecol = jnp.minimum(t_r // 128, MAXTILE)
        half = (pos // RG) * RG
        mask = (
            (iota // RG == pos // RG)
            & (tvec // 128 == tilecol)
            & (tvec != PAD)
            & (tvec < ALIGNED_COLS)
        )
        idx0 = jnp.full((L,), 0, jnp.int32) + r
        idx1 = iota - half
        idx2 = tvec % 128
        plsc.store_scatter(stile, [idx0, idx1, idx2], one16, mask=mask)
        return carry

    lax.fori_loop(0, ROWS_PER_W, build, 0)

    def tail(g, carry):
        tvec = ids_v[pl.ds((g // 2) * L, L)]
        half = (g % 2) * RG
        mask = (iota // RG == g % 2) & (tvec >= ALIGNED_COLS)
        idx0 = jnp.full((L,), 0, jnp.int32) + g
        idx1 = iota - half
        idx2 = tvec - ALIGNED_COLS
        plsc.store_scatter(ttile, [idx0, idx1, idx2], one16, mask=mask)
        return carry

    lax.fori_loop(0, NG, tail, 0)

    def drain(k, carry):
        src, dst = fill_slices(k)
        pltpu.make_async_copy(src, dst, sem).wait()
        return carry

    lax.fori_loop(0, NFILL, drain, 0)

    def spike_slices(r):
        tvec = ids_v[pl.ds((r // L) * L, L)]
        t_r = _extract(tvec, r % L)
        colbase = pl.multiple_of(jnp.minimum(t_r // 128, MAXTILE) * 128, 128)
        rows = pl.ds(pl.multiple_of(base_row + (r // RG) * RG, RG), RG)
        return stile.at[r], out_hbm.at[rows, pl.ds(colbase, 128)]

    def tail_slices(g):
        rows = pl.ds(pl.multiple_of(base_row + g * RG, RG), RG)
        return ttile.at[g], out_hbm.at[rows, pl.ds(ALIGNED_COLS, 32)]

    def sfire(r, carry):
        src, dst = spike_slices(r)
        pltpu.async_copy(src, dst, sem)
        return carry

    lax.fori_loop(0, ROWS_PER_W, sfire, 0)

    def tfire(g, carry):
        src, dst = tail_slices(g)
        pltpu.async_copy(src, dst, sem)
        return carry

    lax.fori_loop(0, NG, tfire, 0)

    def swait(r, carry):
        src, dst = spike_slices(r)
        pltpu.make_async_copy(src, dst, sem).wait()
        return carry

    lax.fori_loop(0, ROWS_PER_W, swait, 0)

    def twait(g, carry):
        src, dst = tail_slices(g)
        pltpu.make_async_copy(src, dst, sem).wait()
        return carry

    lax.fori_loop(0, NG, twait, 0)


@jax.jit
def kernel(trg_token_ids_batch):
    ids = trg_token_ids_batch.reshape(BATCH)
    return _sc_onehot(ids)


# speedup vs baseline: 5.3813x; 2.4648x over previous
import functools

import jax
import jax.numpy as jnp
from jax import lax
from jax.experimental import pallas as pl
from jax.experimental.pallas import tpu as pltpu
from jax.experimental.pallas import tpu_sc as plsc

PAD = 0
VOCAB = 100000
BATCH = 1024
NC, NS, L = 2, 16, 16
NW = NC * NS
VROWS_W = 3120
ZROWS = 104
NFILL_A = VROWS_W // ZROWS
NFILL_B = 31
REM_ROWS = 56

_mesh = plsc.VectorSubcoreMesh(core_axis_name="c", subcore_axis_name="s")


@functools.partial(
    pl.kernel,
    out_type=jax.ShapeDtypeStruct((VOCAB, BATCH), jnp.float32),
    mesh=_mesh,
    scratch_types=[
        pltpu.VMEM((ZROWS, BATCH), jnp.float32),
        pltpu.VMEM((8, 128), jnp.float32),
        pltpu.VMEM((BATCH,), jnp.int32),
        pltpu.SemaphoreType.DMA,
    ],
    compiler_params=pltpu.CompilerParams(
        use_tc_tiling_on_sc=True, needs_layout_passes=False
    ),
)
def _sc_onehot_t(ids_hbm, out_hbm, zbuf, stile, ids_v, sem):
    cid = lax.axis_index("c")
    sid = lax.axis_index("s")
    wid = cid * NS + sid
    is31 = wid == NW - 1
    vlo = wid * VROWS_W
    vhi = jnp.where(is31, VOCAB, vlo + VROWS_W)

    iota = lax.iota(jnp.int32, L)
    zero16 = jnp.zeros((L,), jnp.float32)

    def zloop(i, carry):
        zbuf[i // 64, pl.ds((i % 64) * L, L)] = zero16
        return carry

    lax.fori_loop(0, ZROWS * BATCH // L, zloop, 0)

    pltpu.sync_copy(ids_hbm, ids_v)

    nfill = jnp.where(is31, NFILL_B, NFILL_A)

    def fill_slices(k):
        rows = pl.ds(pl.multiple_of(vlo + k * ZROWS, 8), ZROWS)
        return zbuf, out_hbm.at[rows, :]

    def fire(k, carry):
        src, dst = fill_slices(k)
        pltpu.async_copy(src, dst, sem)
        return carry

    lax.fori_loop(0, nfill, fire, 0)

    tail_rows = pl.ds(pl.multiple_of(vlo + NFILL_B * ZROWS, 8), REM_ROWS)

    @pl.when(is31)
    def _():
        pltpu.async_copy(zbuf.at[pl.ds(0, REM_ROWS), :], out_hbm.at[tail_rows, :], sem)

    def drain(k, carry):
        src, dst = fill_slices(k)
        pltpu.make_async_copy(src, dst, sem).wait()
        return carry

    lax.fori_loop(0, nfill, drain, 0)

    @pl.when(is31)
    def _():
        pltpu.make_async_copy(
            zbuf.at[pl.ds(0, REM_ROWS), :], out_hbm.at[tail_rows, :], sem
        ).wait()

    def _extract(tvec, pos):
        return jnp.sum(jnp.where(iota == pos, tvec, 0))

    def spike(b, carry):
        tvec = ids_v[pl.ds((b // L) * L, L)]
        t = _extract(tvec, b % L)

        @pl.when((t >= vlo) & (t < vhi))
        def _():
            vbase = pl.multiple_of((t // 8) * 8, 8)
            bbase = pl.multiple_of((b // 128) * 128, 128)
            for ch in range(128 // L):
                tv = ids_v[pl.ds(bbase + ch * L, L)]
                nz = tv != PAD
                for q in range(8):
                    hit = (tv == vbase + q) & nz
                    stile[q, pl.ds(ch * L, L)] = jnp.where(hit, 1.0, 0.0).astype(
                        jnp.float32
                    )
            pltpu.async_copy(
                stile,
                out_hbm.at[pl.ds(vbase, 8), pl.ds(bbase, 128)],
                sem,
            )
            pltpu.make_async_copy(
                stile,
                out_hbm.at[pl.ds(vbase, 8), pl.ds(bbase, 128)],
                sem,
            ).wait()

        return carry

    lax.fori_loop(0, BATCH, spike, 0)


def _impl(trg_token_ids_batch):
    ids = trg_token_ids_batch.reshape(BATCH)
    out_t = _sc_onehot_t(ids)
    return out_t.T


kernel = jax.jit(_impl)


# speedup vs baseline: 6.9066x; 1.2834x over previous
import functools

import jax
import jax.numpy as jnp
from jax import lax
from jax.experimental import pallas as pl
from jax.experimental.pallas import tpu as pltpu
from jax.experimental.pallas import tpu_sc as plsc

PAD = 0
VOCAB = 100000
BATCH = 1024
NC, NS, L = 2, 16, 16
NW = NC * NS
VROWS_W = 3120
ZROWS = 48
NFILL_A = VROWS_W // ZROWS
NFILL_B = 68
REM_ROWS = 16
POOL = 76

_mesh = plsc.VectorSubcoreMesh(core_axis_name="c", subcore_axis_name="s")


@functools.partial(
    pl.kernel,
    out_type=jax.ShapeDtypeStruct((VOCAB, BATCH), jnp.float32),
    mesh=_mesh,
    scratch_types=[
        pltpu.VMEM((ZROWS, BATCH), jnp.float32),
        pltpu.VMEM((POOL, 8, 128), jnp.float32),
        pltpu.VMEM((1, 8, 128), jnp.float32),
        pltpu.VMEM((BATCH,), jnp.int32),
        pltpu.VMEM((80,), jnp.int32),
        pltpu.SemaphoreType.DMA,
    ],
    compiler_params=pltpu.CompilerParams(
        use_tc_tiling_on_sc=True, needs_layout_passes=False
    ),
)
def _sc_onehot_t(ids_hbm, out_hbm, zbuf, pool, stile, ids_v, blist, sem):
    cid = lax.axis_index("c")
    sid = lax.axis_index("s")
    wid = cid * NS + sid
    is31 = wid == NW - 1
    vlo = wid * VROWS_W
    vhi = jnp.where(is31, VOCAB, vlo + VROWS_W)

    iota = lax.iota(jnp.int32, L)
    zero16 = jnp.zeros((L,), jnp.float32)

    def zloop(i, carry):
        zbuf[i // 64, pl.ds((i % 64) * L, L)] = zero16
        return carry

    lax.fori_loop(0, ZROWS * BATCH // L, zloop, 0)

    pltpu.sync_copy(ids_hbm, ids_v)

    nfill = jnp.where(is31, NFILL_B, NFILL_A)

    def fill_slices(k):
        rows = pl.ds(pl.multiple_of(vlo + k * ZROWS, 8), ZROWS)
        return zbuf, out_hbm.at[rows, :]

    def fire(k, carry):
        src, dst = fill_slices(k)
        pltpu.async_copy(src, dst, sem)
        return carry

    lax.fori_loop(0, nfill, fire, 0)

    tail_rows = pl.ds(pl.multiple_of(vlo + NFILL_B * ZROWS, 8), REM_ROWS)

    @pl.when(is31)
    def _():
        pltpu.async_copy(
            zbuf.at[pl.ds(0, REM_ROWS), :], out_hbm.at[tail_rows, :], sem
        )

    def _extract(tvec, pos):
        return jnp.sum(jnp.where(iota == pos, tvec, 0))

    def _build(buf, slot, t, b):
        vbase = (t // 8) * 8
        bbase = (b // 128) * 128
        for ch in range(128 // L):
            tv = ids_v[pl.ds(bbase + ch * L, L)]
            nz = tv != PAD
            for q in range(8):
                hit = (tv == vbase + q) & nz
                buf[slot, q, pl.ds(ch * L, L)] = jnp.where(
                    hit, 1.0, 0.0
                ).astype(jnp.float32)

    def _dst(t, b):
        vbase = pl.multiple_of((t // 8) * 8, 8)
        bbase = pl.multiple_of((b // 128) * 128, 128)
        return out_hbm.at[pl.ds(vbase, 8), pl.ds(bbase, 128)]

    def stage(b, carry):
        cnt, inr_total = carry
        tvec = ids_v[pl.ds((b // L) * L, L)]
        t = _extract(tvec, b % L)
        inr = (t >= vlo) & (t < vhi)
        can = inr & (cnt < POOL)

        @pl.when(can)
        def _():
            _build(pool, cnt, t, b)
            plsc.store_scatter(
                blist, [iota * 0 + cnt], iota * 0 + b, mask=iota == 0
            )

        return (
            cnt + jnp.where(can, 1, 0),
            inr_total + jnp.where(inr, 1, 0),
        )

    cnt, inr_total = lax.fori_loop(0, BATCH, stage, (0, 0))

    def drain(k, carry):
        src, dst = fill_slices(k)
        pltpu.make_async_copy(src, dst, sem).wait()
        return carry

    lax.fori_loop(0, nfill, drain, 0)

    @pl.when(is31)
    def _():
        pltpu.make_async_copy(
            zbuf.at[pl.ds(0, REM_ROWS), :], out_hbm.at[tail_rows, :], sem
        ).wait()

    def sfire(k, carry):
        bvec = plsc.load_gather(blist, [iota * 0 + k])
        b_k = _extract(bvec, 0)
        tvec = plsc.load_gather(ids_v, [iota * 0 + b_k])
        t_k = _extract(tvec, 0)
        pltpu.async_copy(pool.at[k], _dst(t_k, b_k), sem)
        return carry

    lax.fori_loop(0, cnt, sfire, 0)

    @pl.when(inr_total > POOL)
    def _():
        def overflow(b, c3):
            tvec = ids_v[pl.ds((b // L) * L, L)]
            t = _extract(tvec, b % L)
            inr = (t >= vlo) & (t < vhi)

            @pl.when(inr & (c3 >= POOL))
            def _():
                _build(stile, 0, t, b)
                pltpu.async_copy(stile.at[0], _dst(t, b), sem)
                pltpu.make_async_copy(stile.at[0], _dst(t, b), sem).wait()

            return c3 + jnp.where(inr, 1, 0)

        lax.fori_loop(0, BATCH, overflow, 0)

    def sdrain(k, carry):
        pltpu.make_async_copy(
            pool.at[k],
            out_hbm.at[pl.ds(pl.multiple_of(vlo, 8), 8), pl.ds(0, 128)],
            sem,
        ).wait()
        return carry

    lax.fori_loop(0, cnt, sdrain, 0)


def _impl(trg_token_ids_batch):
    ids = trg_token_ids_batch.reshape(BATCH)
    out_t = _sc_onehot_t(ids)
    return out_t.T


kernel = jax.jit(_impl)
